# baseline (speedup 1.0000x reference)
import functools

import jax
import jax.numpy as jnp
from jax import lax
from jax.experimental import pallas as pl
from jax.experimental.pallas import tpu as pltpu
from jax.experimental.pallas import tpu_sc as plsc

N_NODES = 10000
N_FEAT = 128
N_HID = 128
N_CLASS = 64

NC, NS = 2, 16
NPAD = 10240
ROWS_PER_SUB = NPAD // NS
NPAD_D = 51200
ROWS_PER_SUB_D = NPAD_D // NS
CHUNK = 128
N_CHUNKS_PER_SUB = 80
E_PER_SUB = CHUNK * N_CHUNKS_PER_SUB
E_PER_CORE = E_PER_SUB * NS
E_TOT = E_PER_CORE * NC
ROWS_PER_CORE = E_PER_CORE // CHUNK

_mesh = plsc.VectorSubcoreMesh(core_axis_name="c", subcore_axis_name="s")


@functools.partial(
    pl.kernel,
    out_type=jax.ShapeDtypeStruct((NC * NPAD_D,), jnp.float32),
    mesh=_mesh,
    scratch_types=[
        pltpu.VMEM((N_CHUNKS_PER_SUB, CHUNK), jnp.int32),
        pltpu.VMEM((CHUNK,), jnp.float32),
        pltpu.VMEM((ROWS_PER_SUB_D,), jnp.float32),
        pltpu.VMEM_SHARED((NPAD_D,), jnp.float32),
    ],
)
def s_deg(dst2d_hbm, out, dsts, ones_v, zbuf, acc):
    c = lax.axis_index("c")
    s = lax.axis_index("s")
    ones = jnp.ones((16,), jnp.float32)
    zeros = jnp.zeros((16,), jnp.float32)

    def fill1(i, carry):
        ones_v[pl.ds(i * 16, 16)] = ones
        return carry

    lax.fori_loop(0, CHUNK // 16, fill1, 0)

    def fill0(i, carry):
        zbuf[pl.ds(i * 16, 16)] = zeros
        return carry

    lax.fori_loop(0, ROWS_PER_SUB_D // 16, fill0, 0)
    pltpu.sync_copy(zbuf, acc.at[pl.ds(s * ROWS_PER_SUB_D, ROWS_PER_SUB_D)])
    row0 = c * ROWS_PER_CORE + s * N_CHUNKS_PER_SUB
    pltpu.sync_copy(dst2d_hbm.at[pl.ds(row0, N_CHUNKS_PER_SUB)], dsts)
    plsc.subcore_barrier()

    def body(i, carry):
        pltpu.sync_copy(ones_v, acc.at[dsts.at[i]], add=True)
        return carry

    lax.fori_loop(0, N_CHUNKS_PER_SUB, body, 0)
    plsc.subcore_barrier()
    rows = pl.ds(s * ROWS_PER_SUB_D, ROWS_PER_SUB_D)
    pltpu.sync_copy(
        acc.at[rows],
        out.at[pl.ds(c * NPAD_D + s * ROWS_PER_SUB_D, ROWS_PER_SUB_D)])


@functools.partial(
    pl.kernel,
    out_type=jax.ShapeDtypeStruct((NC, NPAD, N_HID), jnp.float32),
    mesh=_mesh,
    scratch_types=[
        pltpu.VMEM((CHUNK,), jnp.int32),
        pltpu.VMEM((CHUNK,), jnp.int32),
        pltpu.VMEM((N_CHUNKS_PER_SUB, CHUNK), jnp.int32),
        pltpu.VMEM((CHUNK, N_HID), jnp.float32),
        pltpu.VMEM((CHUNK, N_HID), jnp.float32),
        pltpu.VMEM_SHARED((NPAD, N_HID), jnp.float32),
        pltpu.SemaphoreType.DMA,
        pltpu.SemaphoreType.DMA,
        pltpu.SemaphoreType.DMA,
        pltpu.SemaphoreType.DMA,
    ],
)
def s_mp(h_hbm, src_hbm, dst2d_hbm, zeros_hbm, out,
         src0, src1, dsts, rows0, rows1, acc, sem0, sem1, ssem0, ssem1):
    c = lax.axis_index("c")
    s = lax.axis_index("s")
    w = c * NS + s
    pltpu.sync_copy(zeros_hbm, acc.at[pl.ds(s * ROWS_PER_SUB, ROWS_PER_SUB)])
    row0 = c * ROWS_PER_CORE + s * N_CHUNKS_PER_SUB
    pltpu.sync_copy(dst2d_hbm.at[pl.ds(row0, N_CHUNKS_PER_SUB)], dsts)
    plsc.subcore_barrier()
    ebase = w * E_PER_SUB

    def fetch(i, src_v, rows_v, sem):
        pltpu.sync_copy(src_hbm.at[pl.ds(ebase + i * CHUNK, CHUNK)], src_v)
        pltpu.async_copy(h_hbm.at[src_v], rows_v, sem)

    def start_scatter(i, src_v, rows_v, gsem, ssem):
        pltpu.make_async_copy(h_hbm.at[src_v], rows_v, gsem).wait()
        pltpu.async_copy(rows_v, acc.at[dsts.at[i]], ssem, add=True)

    def wait_scatter(i, rows_v, ssem):
        pltpu.make_async_copy(rows_v, acc.at[dsts.at[i]], ssem).wait()

    fetch(0, src0, rows0, sem0)
    fetch(1, src1, rows1, sem1)
    start_scatter(0, src0, rows0, sem0, ssem0)

    def body(k, carry):
        start_scatter(2 * k + 1, src1, rows1, sem1, ssem1)
        wait_scatter(2 * k, rows0, ssem0)
        fetch(2 * k + 2, src0, rows0, sem0)
        start_scatter(2 * k + 2, src0, rows0, sem0, ssem0)
        wait_scatter(2 * k + 1, rows1, ssem1)
        fetch(2 * k + 3, src1, rows1, sem1)
        return carry

    lax.fori_loop(0, N_CHUNKS_PER_SUB // 2 - 1, body, 0)
    start_scatter(N_CHUNKS_PER_SUB - 1, src1, rows1, sem1, ssem1)
    wait_scatter(N_CHUNKS_PER_SUB - 2, rows0, ssem0)
    wait_scatter(N_CHUNKS_PER_SUB - 1, rows1, ssem1)

    plsc.subcore_barrier()
    rows = pl.ds(s * ROWS_PER_SUB, ROWS_PER_SUB)
    pltpu.sync_copy(acc.at[rows], out.at[c, rows])


ROW_BLK = 400
N_BLKS = N_NODES // ROW_BLK
PART1_BLK_OFF = NPAD_D // ROW_BLK


def _dinv(d0, d1):
    return lax.rsqrt(d0[:, :1] + d1[:, :1] + 1.0)


def _p1_body(x_ref, w_ref, d0_ref, d1_ref, o_ref):
    dinv = _dinv(d0_ref[...], d1_ref[...])
    o_ref[...] = jnp.dot(x_ref[...], w_ref[...],
                         preferred_element_type=jnp.float32) * dinv


def p1(x, w1, deg):
    return pl.pallas_call(
        _p1_body,
        grid=(N_BLKS,),
        in_specs=[
            pl.BlockSpec((ROW_BLK, N_FEAT), lambda i: (i, 0)),
            pl.BlockSpec((N_FEAT, N_HID), lambda i: (0, 0)),
            pl.BlockSpec((ROW_BLK, 1), lambda i: (i, 0)),
            pl.BlockSpec((ROW_BLK, 1), lambda i: (i + PART1_BLK_OFF, 0)),
        ],
        out_specs=pl.BlockSpec((ROW_BLK, N_HID), lambda i: (i, 0)),
        out_shape=jax.ShapeDtypeStruct((N_NODES, N_HID), jnp.float32),
    )(x, w1, deg, deg)


def _p2_body(m_ref0, m_ref1, hs_ref, d0_ref, d1_ref, b_ref, w_ref, o_ref):
    dinv = _dinv(d0_ref[...], d1_ref[...])
    agg = (m_ref0[0] + m_ref1[0] + hs_ref[...]) * dinv
    h = jnp.maximum(agg + b_ref[...], 0.0)
    o_ref[...] = jnp.dot(h, w_ref[...],
                         preferred_element_type=jnp.float32) * dinv


def p2(parts, hs, deg, b1, w2):
    return pl.pallas_call(
        _p2_body,
        grid=(N_BLKS,),
        in_specs=[
            pl.BlockSpec((1, ROW_BLK, N_HID), lambda i: (0, i, 0)),
            pl.BlockSpec((1, ROW_BLK, N_HID), lambda i: (1, i, 0)),
            pl.BlockSpec((ROW_BLK, N_HID), lambda i: (i, 0)),
            pl.BlockSpec((ROW_BLK, 1), lambda i: (i, 0)),
            pl.BlockSpec((ROW_BLK, 1), lambda i: (i + PART1_BLK_OFF, 0)),
            pl.BlockSpec((1, N_HID), lambda i: (0, 0)),
            pl.BlockSpec((N_HID, N_HID), lambda i: (0, 0)),
        ],
        out_specs=pl.BlockSpec((ROW_BLK, N_HID), lambda i: (i, 0)),
        out_shape=jax.ShapeDtypeStruct((N_NODES, N_HID), jnp.float32),
    )(parts, parts, hs, deg, deg, b1, w2)


def _p3_body(m_ref0, m_ref1, hs_ref, d0_ref, d1_ref, b_ref, w_ref, b3_ref,
             o_ref):
    dinv = _dinv(d0_ref[...], d1_ref[...])
    agg = (m_ref0[0] + m_ref1[0] + hs_ref[...]) * dinv
    h = jnp.maximum(agg + b_ref[...], 0.0)
    o_ref[...] = jnp.dot(h, w_ref[...],
                         preferred_element_type=jnp.float32) + b3_ref[...]


def p3(parts, hs, deg, b2, w3, b3):
    return pl.pallas_call(
        _p3_body,
        grid=(N_BLKS,),
        in_specs=[
            pl.BlockSpec((1, ROW_BLK, N_HID), lambda i: (0, i, 0)),
            pl.BlockSpec((1, ROW_BLK, N_HID), lambda i: (1, i, 0)),
            pl.BlockSpec((ROW_BLK, N_HID), lambda i: (i, 0)),
            pl.BlockSpec((ROW_BLK, 1), lambda i: (i, 0)),
            pl.BlockSpec((ROW_BLK, 1), lambda i: (i + PART1_BLK_OFF, 0)),
            pl.BlockSpec((1, N_HID), lambda i: (0, 0)),
            pl.BlockSpec((N_HID, N_CLASS), lambda i: (0, 0)),
            pl.BlockSpec((1, N_CLASS), lambda i: (0, 0)),
        ],
        out_specs=pl.BlockSpec((ROW_BLK, N_CLASS), lambda i: (i, 0)),
        out_shape=jax.ShapeDtypeStruct((N_NODES, N_CLASS), jnp.float32),
    )(parts, parts, hs, deg, deg, b2, w3, b3)


def kernel(x, edge_index, W1, b1, W2, b2, W3, b3):
    ei = edge_index.astype(jnp.int32)
    pad_n = E_TOT - ei.shape[1]
    pad_i = jnp.arange(pad_n, dtype=jnp.int32)
    src = jnp.concatenate([ei[0], pad_i % N_NODES])
    dst = jnp.concatenate([ei[1], N_NODES + pad_i % (NPAD - N_NODES)])
    dst2d = dst.reshape(-1, CHUNK)

    zerosH = jnp.zeros((ROWS_PER_SUB, N_HID), jnp.float32)
    deg = s_deg(dst2d).reshape(NC * NPAD_D, 1)
    hs0 = p1(x, W1, deg)
    m = s_mp(hs0, src, dst2d, zerosH)
    hs1 = p2(m, hs0, deg, b1.reshape(1, N_HID), W2)
    n = s_mp(hs1, src, dst2d, zerosH)
    out = p3(n, hs1, deg, b2.reshape(1, N_HID), W3, b3.reshape(1, N_CLASS))
    return out


# speedup vs baseline: 1.2396x; 1.2396x over previous
import functools

import jax
import jax.numpy as jnp
from jax import lax
from jax.experimental import pallas as pl
from jax.experimental.pallas import tpu as pltpu
from jax.experimental.pallas import tpu_sc as plsc

N_NODES = 10000
N_FEAT = 128
N_HID = 128
N_CLASS = 64

NC, NS = 2, 16
NPAD = 10240
ROWS_PER_SUB = NPAD // NS
NPAD_D = 51200
ROWS_PER_SUB_D = NPAD_D // NS
CHUNK = 128
N_CHUNKS_PER_SUB = 80
E_PER_SUB = CHUNK * N_CHUNKS_PER_SUB
E_PER_CORE = E_PER_SUB * NS
E_TOT = E_PER_CORE * NC
ROWS_PER_CORE = E_PER_CORE // CHUNK

_mesh = plsc.VectorSubcoreMesh(core_axis_name="c", subcore_axis_name="s")


@functools.partial(
    pl.kernel,
    out_type=jax.ShapeDtypeStruct((NC * NPAD_D,), jnp.float32),
    mesh=_mesh,
    scratch_types=[
        pltpu.VMEM((N_CHUNKS_PER_SUB, CHUNK), jnp.int32),
        pltpu.VMEM((CHUNK,), jnp.float32),
        pltpu.VMEM((ROWS_PER_SUB_D,), jnp.float32),
        pltpu.VMEM_SHARED((NPAD_D,), jnp.float32),
    ],
)
def s_deg(dst2d_hbm, out, dsts, ones_v, zbuf, acc):
    c = lax.axis_index("c")
    s = lax.axis_index("s")
    ones = jnp.ones((16,), jnp.float32)
    zeros = jnp.zeros((16,), jnp.float32)

    def fill1(i, carry):
        ones_v[pl.ds(i * 16, 16)] = ones
        return carry

    lax.fori_loop(0, CHUNK // 16, fill1, 0)

    def fill0(i, carry):
        zbuf[pl.ds(i * 16, 16)] = zeros
        return carry

    lax.fori_loop(0, ROWS_PER_SUB_D // 16, fill0, 0)
    pltpu.sync_copy(zbuf, acc.at[pl.ds(s * ROWS_PER_SUB_D, ROWS_PER_SUB_D)])
    row0 = c * ROWS_PER_CORE + s * N_CHUNKS_PER_SUB
    pltpu.sync_copy(dst2d_hbm.at[pl.ds(row0, N_CHUNKS_PER_SUB)], dsts)
    plsc.subcore_barrier()

    def body(i, carry):
        pltpu.sync_copy(ones_v, acc.at[dsts.at[i]], add=True)
        return carry

    lax.fori_loop(0, N_CHUNKS_PER_SUB, body, 0)
    plsc.subcore_barrier()
    rows = pl.ds(s * ROWS_PER_SUB_D, ROWS_PER_SUB_D)
    pltpu.sync_copy(
        acc.at[rows],
        out.at[pl.ds(c * NPAD_D + s * ROWS_PER_SUB_D, ROWS_PER_SUB_D)])


@functools.partial(
    pl.kernel,
    out_type=jax.ShapeDtypeStruct((NC, NPAD, N_HID), jnp.float32),
    mesh=_mesh,
    scratch_types=[
        pltpu.VMEM((CHUNK,), jnp.int32),
        pltpu.VMEM((CHUNK,), jnp.int32),
        pltpu.VMEM((N_CHUNKS_PER_SUB, CHUNK), jnp.int32),
        pltpu.VMEM((CHUNK, N_HID), jnp.float32),
        pltpu.VMEM((CHUNK, N_HID), jnp.float32),
        pltpu.VMEM_SHARED((NPAD, N_HID), jnp.float32),
        pltpu.SemaphoreType.DMA,
        pltpu.SemaphoreType.DMA,
    ],
)
def s_mp(h_hbm, src_hbm, dst2d_hbm, out,
         src0, src1, dsts, rows0, rows1, acc, sem0, sem1):
    c = lax.axis_index("c")
    s = lax.axis_index("s")
    w = c * NS + s
    zeros = jnp.zeros((16,), jnp.float32)

    def fill0(i, carry):
        def fill0j(j, carry2):
            rows0[i, pl.ds(j * 16, 16)] = zeros
            return carry2
        return lax.fori_loop(0, N_HID // 16, fill0j, carry)

    lax.fori_loop(0, CHUNK, fill0, 0)
    for r in range(ROWS_PER_SUB // CHUNK):
        pltpu.sync_copy(
            rows0, acc.at[pl.ds(s * ROWS_PER_SUB + r * CHUNK, CHUNK)])
    row0 = c * ROWS_PER_CORE + s * N_CHUNKS_PER_SUB
    pltpu.sync_copy(dst2d_hbm.at[pl.ds(row0, N_CHUNKS_PER_SUB)], dsts)
    plsc.subcore_barrier()
    ebase = w * E_PER_SUB

    def fetch(i, src_v, rows_v, sem):
        pltpu.sync_copy(src_hbm.at[pl.ds(ebase + i * CHUNK, CHUNK)], src_v)
        pltpu.async_copy(h_hbm.at[src_v], rows_v, sem)

    def drain_scatter(i, src_v, rows_v, sem):
        pltpu.make_async_copy(h_hbm.at[src_v], rows_v, sem).wait()
        pltpu.sync_copy(rows_v, acc.at[dsts.at[i]], add=True)

    fetch(0, src0, rows0, sem0)
    fetch(1, src1, rows1, sem1)

    def body(k, carry):
        drain_scatter(2 * k, src0, rows0, sem0)
        fetch(2 * k + 2, src0, rows0, sem0)
        drain_scatter(2 * k + 1, src1, rows1, sem1)
        fetch(2 * k + 3, src1, rows1, sem1)
        return carry

    lax.fori_loop(0, N_CHUNKS_PER_SUB // 2 - 1, body, 0)
    drain_scatter(N_CHUNKS_PER_SUB - 2, src0, rows0, sem0)
    drain_scatter(N_CHUNKS_PER_SUB - 1, src1, rows1, sem1)

    plsc.subcore_barrier()
    rows = pl.ds(s * ROWS_PER_SUB, ROWS_PER_SUB)
    pltpu.sync_copy(acc.at[rows], out.at[c, rows])


ROW_BLK = 400
N_BLKS = N_NODES // ROW_BLK
PART1_BLK_OFF = NPAD_D // ROW_BLK


def _dinv(d0, d1):
    return lax.rsqrt(d0[:, :1] + d1[:, :1] + 1.0)


def _p1_body(x_ref, w_ref, d0_ref, d1_ref, o_ref):
    dinv = _dinv(d0_ref[...], d1_ref[...])
    o_ref[...] = jnp.dot(x_ref[...], w_ref[...],
                         preferred_element_type=jnp.float32) * dinv


def p1(x, w1, deg):
    return pl.pallas_call(
        _p1_body,
        grid=(N_BLKS,),
        in_specs=[
            pl.BlockSpec((ROW_BLK, N_FEAT), lambda i: (i, 0)),
            pl.BlockSpec((N_FEAT, N_HID), lambda i: (0, 0)),
            pl.BlockSpec((ROW_BLK, 1), lambda i: (i, 0)),
            pl.BlockSpec((ROW_BLK, 1), lambda i: (i + PART1_BLK_OFF, 0)),
        ],
        out_specs=pl.BlockSpec((ROW_BLK, N_HID), lambda i: (i, 0)),
        out_shape=jax.ShapeDtypeStruct((N_NODES, N_HID), jnp.float32),
    )(x, w1, deg, deg)


def _p2_body(m_ref0, m_ref1, hs_ref, d0_ref, d1_ref, b_ref, w_ref, o_ref):
    dinv = _dinv(d0_ref[...], d1_ref[...])
    agg = (m_ref0[0] + m_ref1[0] + hs_ref[...]) * dinv
    h = jnp.maximum(agg + b_ref[...], 0.0)
    o_ref[...] = jnp.dot(h, w_ref[...],
                         preferred_element_type=jnp.float32) * dinv


def p2(parts, hs, deg, b1, w2):
    return pl.pallas_call(
        _p2_body,
        grid=(N_BLKS,),
        in_specs=[
            pl.BlockSpec((1, ROW_BLK, N_HID), lambda i: (0, i, 0)),
            pl.BlockSpec((1, ROW_BLK, N_HID), lambda i: (1, i, 0)),
            pl.BlockSpec((ROW_BLK, N_HID), lambda i: (i, 0)),
            pl.BlockSpec((ROW_BLK, 1), lambda i: (i, 0)),
            pl.BlockSpec((ROW_BLK, 1), lambda i: (i + PART1_BLK_OFF, 0)),
            pl.BlockSpec((1, N_HID), lambda i: (0, 0)),
            pl.BlockSpec((N_HID, N_HID), lambda i: (0, 0)),
        ],
        out_specs=pl.BlockSpec((ROW_BLK, N_HID), lambda i: (i, 0)),
        out_shape=jax.ShapeDtypeStruct((N_NODES, N_HID), jnp.float32),
    )(parts, parts, hs, deg, deg, b1, w2)


def _p3_body(m_ref0, m_ref1, hs_ref, d0_ref, d1_ref, b_ref, w_ref, b3_ref,
             o_ref):
    dinv = _dinv(d0_ref[...], d1_ref[...])
    agg = (m_ref0[0] + m_ref1[0] + hs_ref[...]) * dinv
    h = jnp.maximum(agg + b_ref[...], 0.0)
    o_ref[...] = jnp.dot(h, w_ref[...],
                         preferred_element_type=jnp.float32) + b3_ref[...]


def p3(parts, hs, deg, b2, w3, b3):
    return pl.pallas_call(
        _p3_body,
        grid=(N_BLKS,),
        in_specs=[
            pl.BlockSpec((1, ROW_BLK, N_HID), lambda i: (0, i, 0)),
            pl.BlockSpec((1, ROW_BLK, N_HID), lambda i: (1, i, 0)),
            pl.BlockSpec((ROW_BLK, N_HID), lambda i: (i, 0)),
            pl.BlockSpec((ROW_BLK, 1), lambda i: (i, 0)),
            pl.BlockSpec((ROW_BLK, 1), lambda i: (i + PART1_BLK_OFF, 0)),
            pl.BlockSpec((1, N_HID), lambda i: (0, 0)),
            pl.BlockSpec((N_HID, N_CLASS), lambda i: (0, 0)),
            pl.BlockSpec((1, N_CLASS), lambda i: (0, 0)),
        ],
        out_specs=pl.BlockSpec((ROW_BLK, N_CLASS), lambda i: (i, 0)),
        out_shape=jax.ShapeDtypeStruct((N_NODES, N_CLASS), jnp.float32),
    )(parts, parts, hs, deg, deg, b2, w3, b3)


def kernel(x, edge_index, W1, b1, W2, b2, W3, b3):
    ei = edge_index.astype(jnp.int32)
    pad_n = E_TOT - ei.shape[1]
    pad_i = jnp.arange(pad_n, dtype=jnp.int32)
    src = jnp.concatenate([ei[0], pad_i % N_NODES])
    dst = jnp.concatenate([ei[1], N_NODES + pad_i % (NPAD - N_NODES)])
    dst2d = dst.reshape(-1, CHUNK)

    deg = s_deg(dst2d).reshape(NC * NPAD_D, 1)
    hs0 = p1(x, W1, deg)
    m = s_mp(hs0, src, dst2d)
    hs1 = p2(m, hs0, deg, b1.reshape(1, N_HID), W2)
    n = s_mp(hs1, src, dst2d)
    out = p3(n, hs1, deg, b2.reshape(1, N_HID), W3, b3.reshape(1, N_CLASS))
    return out
